# baseline (device time: 53850 ns/iter reference)
import jax
import jax.numpy as jnp
from jax import lax
from jax.experimental import pallas as pl
from jax.experimental.pallas import tpu as pltpu

T = 1024
D = 1024
V_LOCAL = 8192
CH = 8
R = T // CH

_sem_signal = getattr(pl, "semaphore_signal", None) or pltpu.semaphore_signal
_sem_wait = getattr(pl, "semaphore_wait", None) or pltpu.semaphore_wait
_CompilerParams = getattr(pltpu, "CompilerParams", None) or pltpu.TPUCompilerParams


def kernel(ids, E):
    my_x = lax.axis_index("x")
    off = my_x * V_LOCAL
    owned = (ids >= off) & (ids < off + V_LOCAL)
    order = jnp.argsort(jnp.logical_not(owned), stable=True)
    loc_s = jnp.clip(ids[order] - off, 0, V_LOCAL - 1).astype(jnp.int32)
    n_own = jnp.sum(owned).astype(jnp.int32)
    scalars = jnp.reshape(n_own, (1,))
    inv = jnp.argsort(order).astype(jnp.int32)

    def body(loc_ref, n_ref, e_ref, out_ref, send_buf, recv_buf,
             gsem, send_sems, recv_sems):
        mx = lax.axis_index("x")
        my = lax.axis_index("y")
        mz = lax.axis_index("z")
        partner = (1 - mx, my, mz)

        n_mine = n_ref[0]
        n_in = T - n_mine
        nch_out = lax.div(n_mine + R - 1, R)
        nch_in = lax.div(n_in + R - 1, R)

        barrier_sem = pltpu.get_barrier_semaphore()
        _sem_signal(barrier_sem, inc=1, device_id=partner,
                    device_id_type=pl.DeviceIdType.MESH)
        _sem_wait(barrier_sem, 1)

        def issue_chunk(c):
            def issue(i, carry):
                pltpu.make_async_copy(
                    e_ref.at[pl.ds(loc_ref[i], 1)], out_ref.at[pl.ds(i, 1)],
                    gsem.at[c],
                ).start()
                return carry

            lax.fori_loop(c * R, (c + 1) * R, issue, 0, unroll=8)

        for c in range(CH):
            @pl.when(c < nch_out)
            def _(c=c):
                issue_chunk(c)

        rdmas = []
        for c in range(CH):
            sl = pl.ds(c * R, R)
            rdma = pltpu.make_async_remote_copy(
                src_ref=send_buf.at[sl],
                dst_ref=recv_buf.at[sl],
                send_sem=send_sems.at[c],
                recv_sem=recv_sems.at[c],
                device_id=partner,
                device_id_type=pl.DeviceIdType.MESH,
            )
            rdmas.append(rdma)

            @pl.when(c < nch_out)
            def _(c=c, sl=sl, rdma=rdma):
                pltpu.make_async_copy(
                    e_ref.at[pl.ds(0, R)], out_ref.at[sl], gsem.at[c]
                ).wait()
                send_buf[sl, :] = out_ref[sl, :].astype(jnp.bfloat16)
                rdma.start()

        for c in range(CH):
            @pl.when(c < nch_in)
            def _(c=c):
                pltpu.make_async_remote_copy(
                    src_ref=send_buf.at[pl.ds(0, R)],
                    dst_ref=recv_buf.at[pl.ds(0, R)],
                    send_sem=send_sems.at[c],
                    recv_sem=recv_sems.at[c],
                    device_id=partner,
                    device_id_type=pl.DeviceIdType.MESH,
                ).wait_recv()

        row = lax.broadcasted_iota(jnp.int32, (T, 1), 0)
        rolled = pltpu.roll(recv_buf[...], n_mine, 0)
        out_ref[...] = jnp.where(
            row < n_mine,
            out_ref[...],
            rolled.astype(jnp.float32),
        )

        for c in range(CH):
            @pl.when(c < nch_out)
            def _(c=c):
                rdmas[c].wait_send()

    out_sorted = pl.pallas_call(
        body,
        out_shape=jax.ShapeDtypeStruct((T, D), jnp.float32),
        in_specs=[
            pl.BlockSpec(memory_space=pltpu.SMEM),
            pl.BlockSpec(memory_space=pltpu.SMEM),
            pl.BlockSpec(memory_space=pl.ANY),
        ],
        out_specs=pl.BlockSpec(memory_space=pltpu.VMEM),
        scratch_shapes=[
            pltpu.VMEM((T, D), jnp.bfloat16),
            pltpu.VMEM((T, D), jnp.bfloat16),
            pltpu.SemaphoreType.DMA((CH,)),
            pltpu.SemaphoreType.DMA((CH,)),
            pltpu.SemaphoreType.DMA((CH,)),
        ],
        compiler_params=_CompilerParams(collective_id=0),
    )(loc_s, scalars, E)

    return jnp.take(out_sorted, inv, axis=0)


# device time: 52057 ns/iter; 1.0344x vs baseline; 1.0344x over previous
import jax
import jax.numpy as jnp
from jax import lax
from jax.experimental import pallas as pl
from jax.experimental.pallas import tpu as pltpu

T = 1024
D = 1024
V_LOCAL = 8192
CH = 8
R = T // CH

_sem_signal = getattr(pl, "semaphore_signal", None) or pltpu.semaphore_signal
_sem_wait = getattr(pl, "semaphore_wait", None) or pltpu.semaphore_wait
_CompilerParams = getattr(pltpu, "CompilerParams", None) or pltpu.TPUCompilerParams


def kernel(ids, E):
    my_x = lax.axis_index("x")
    off = my_x * V_LOCAL
    owned = (ids >= off) & (ids < off + V_LOCAL)
    order = jnp.argsort(jnp.logical_not(owned), stable=True)
    loc_s = jnp.clip(ids[order] - off, 0, V_LOCAL - 1).astype(jnp.int32)
    ow = owned.astype(jnp.int32)
    cum = jnp.cumsum(ow)
    n_own = cum[-1]
    cumn = jnp.cumsum(1 - ow)
    pos = jnp.where(owned, cum - 1, n_own + cumn - 1).astype(jnp.int32)
    scalars = jnp.reshape(n_own, (1,))

    def body(loc_ref, n_ref, e_ref, out_ref, send_buf, recv_buf,
             gsem, send_sems, recv_sems):
        mx = lax.axis_index("x")
        my = lax.axis_index("y")
        mz = lax.axis_index("z")
        partner = (1 - mx, my, mz)

        n_mine = n_ref[0]
        n_in = T - n_mine
        nch_out = lax.div(n_mine + R - 1, R)
        nch_in = lax.div(n_in + R - 1, R)

        barrier_sem = pltpu.get_barrier_semaphore()
        _sem_signal(barrier_sem, inc=1, device_id=partner,
                    device_id_type=pl.DeviceIdType.MESH)
        _sem_wait(barrier_sem, 1)

        def issue_chunk(c):
            def issue(i, carry):
                pltpu.make_async_copy(
                    e_ref.at[pl.ds(loc_ref[i], 1)], out_ref.at[pl.ds(i, 1)],
                    gsem.at[c],
                ).start()
                return carry

            lax.fori_loop(c * R, (c + 1) * R, issue, 0, unroll=8)

        for c in range(CH):
            @pl.when(c < nch_out)
            def _(c=c):
                issue_chunk(c)

        rdmas = []
        for c in range(CH):
            sl = pl.ds(c * R, R)
            rdma = pltpu.make_async_remote_copy(
                src_ref=send_buf.at[sl],
                dst_ref=recv_buf.at[sl],
                send_sem=send_sems.at[c],
                recv_sem=recv_sems.at[c],
                device_id=partner,
                device_id_type=pl.DeviceIdType.MESH,
            )
            rdmas.append(rdma)

            @pl.when(c < nch_out)
            def _(c=c, sl=sl, rdma=rdma):
                pltpu.make_async_copy(
                    e_ref.at[pl.ds(0, R)], out_ref.at[sl], gsem.at[c]
                ).wait()
                send_buf[sl, :] = out_ref[sl, :].astype(jnp.bfloat16)
                rdma.start()

        for c in range(CH):
            @pl.when(c < nch_in)
            def _(c=c):
                pltpu.make_async_remote_copy(
                    src_ref=send_buf.at[pl.ds(0, R)],
                    dst_ref=recv_buf.at[pl.ds(0, R)],
                    send_sem=send_sems.at[c],
                    recv_sem=recv_sems.at[c],
                    device_id=partner,
                    device_id_type=pl.DeviceIdType.MESH,
                ).wait_recv()

        row = lax.broadcasted_iota(jnp.int32, (T, 1), 0)
        rolled = pltpu.roll(recv_buf[...], n_mine, 0)
        out_ref[...] = jnp.where(
            row < n_mine,
            out_ref[...],
            rolled.astype(jnp.float32),
        )

        for c in range(CH):
            @pl.when(c < nch_out)
            def _(c=c):
                rdmas[c].wait_send()

    out_sorted = pl.pallas_call(
        body,
        out_shape=jax.ShapeDtypeStruct((T, D), jnp.float32),
        in_specs=[
            pl.BlockSpec(memory_space=pltpu.SMEM),
            pl.BlockSpec(memory_space=pltpu.SMEM),
            pl.BlockSpec(memory_space=pl.ANY),
        ],
        out_specs=pl.BlockSpec(memory_space=pltpu.VMEM),
        scratch_shapes=[
            pltpu.VMEM((T, D), jnp.bfloat16),
            pltpu.VMEM((T, D), jnp.bfloat16),
            pltpu.SemaphoreType.DMA((CH,)),
            pltpu.SemaphoreType.DMA((CH,)),
            pltpu.SemaphoreType.DMA((CH,)),
        ],
        compiler_params=_CompilerParams(collective_id=0),
    )(loc_s, scalars, E)

    return jnp.take(out_sorted, pos, axis=0)


# device time: 35917 ns/iter; 1.4993x vs baseline; 1.4494x over previous
import jax
import jax.numpy as jnp
from jax import lax
from jax.experimental import pallas as pl
from jax.experimental.pallas import tpu as pltpu

T = 1024
D = 1024
V_LOCAL = 8192
CH = 8
R = T // CH

_sem_signal = getattr(pl, "semaphore_signal", None) or pltpu.semaphore_signal
_sem_wait = getattr(pl, "semaphore_wait", None) or pltpu.semaphore_wait
_CompilerParams = getattr(pltpu, "CompilerParams", None) or pltpu.TPUCompilerParams


def kernel(ids, E):
    my_x = lax.axis_index("x")
    off = my_x * V_LOCAL
    owned = (ids >= off) & (ids < off + V_LOCAL)
    ow = owned.astype(jnp.int32)
    loc = jnp.clip(ids - off, 0, V_LOCAL - 1).astype(jnp.int32)
    cnt = jnp.sum(ow.reshape(CH, R), axis=1).astype(jnp.int32)
    mask = owned.astype(jnp.float32)[:, None]

    def body(loc_ref, ow_ref, cnt_ref, mask_ref, e_ref, out_ref,
             send_buf, recv_buf, gsem, send_sems, recv_sems):
        mx = lax.axis_index("x")
        my = lax.axis_index("y")
        mz = lax.axis_index("z")
        partner = (1 - mx, my, mz)

        barrier_sem = pltpu.get_barrier_semaphore()
        _sem_signal(barrier_sem, inc=1, device_id=partner,
                    device_id_type=pl.DeviceIdType.MESH,)
        _sem_wait(barrier_sem, 1)

        def issue_chunk(c):
            def issue(i, carry):
                @pl.when(ow_ref[i] == 1)
                def _():
                    pltpu.make_async_copy(
                        e_ref.at[pl.ds(loc_ref[i], 1)],
                        out_ref.at[pl.ds(i, 1)],
                        gsem.at[c],
                    ).start()
                return carry

            lax.fori_loop(c * R, (c + 1) * R, issue, 0, unroll=8)

        issue_chunk(0)
        issue_chunk(1)

        rdmas = []
        for c in range(CH):
            def drain(j, carry, c=c):
                pltpu.make_async_copy(
                    e_ref.at[pl.ds(0, 1)], out_ref.at[pl.ds(0, 1)],
                    gsem.at[c],
                ).wait()
                return carry

            lax.fori_loop(0, cnt_ref[c], drain, 0)

            sl = pl.ds(c * R, R)
            m = jnp.where(mask_ref[sl, :] > 0.0, out_ref[sl, :], 0.0)
            out_ref[sl, :] = m
            send_buf[sl, :] = m.astype(jnp.bfloat16)
            rdma = pltpu.make_async_remote_copy(
                src_ref=send_buf.at[sl],
                dst_ref=recv_buf.at[sl],
                send_sem=send_sems.at[c],
                recv_sem=recv_sems.at[c],
                device_id=partner,
                device_id_type=pl.DeviceIdType.MESH,
            )
            rdma.start()
            rdmas.append(rdma)
            if c + 2 < CH:
                issue_chunk(c + 2)

        for c in range(CH):
            rdmas[c].wait_recv()
            sl = pl.ds(c * R, R)
            out_ref[sl, :] = out_ref[sl, :] + recv_buf[sl, :].astype(jnp.float32)

        for c in range(CH):
            rdmas[c].wait_send()

    return pl.pallas_call(
        body,
        out_shape=jax.ShapeDtypeStruct((T, D), jnp.float32),
        in_specs=[
            pl.BlockSpec(memory_space=pltpu.SMEM),
            pl.BlockSpec(memory_space=pltpu.SMEM),
            pl.BlockSpec(memory_space=pltpu.SMEM),
            pl.BlockSpec(memory_space=pltpu.VMEM),
            pl.BlockSpec(memory_space=pl.ANY),
        ],
        out_specs=pl.BlockSpec(memory_space=pltpu.VMEM),
        scratch_shapes=[
            pltpu.VMEM((T, D), jnp.bfloat16),
            pltpu.VMEM((T, D), jnp.bfloat16),
            pltpu.SemaphoreType.DMA((CH,)),
            pltpu.SemaphoreType.DMA((CH,)),
            pltpu.SemaphoreType.DMA((CH,)),
        ],
        compiler_params=_CompilerParams(collective_id=0),
    )(loc, ow, cnt, mask, E)


# device time: 35596 ns/iter; 1.5128x vs baseline; 1.0090x over previous
import jax
import jax.numpy as jnp
from jax import lax
from jax.experimental import pallas as pl
from jax.experimental.pallas import tpu as pltpu

T = 1024
D = 1024
V_LOCAL = 8192
CH = 8
R = T // CH

_sem_signal = getattr(pl, "semaphore_signal", None) or pltpu.semaphore_signal
_sem_wait = getattr(pl, "semaphore_wait", None) or pltpu.semaphore_wait
_CompilerParams = getattr(pltpu, "CompilerParams", None) or pltpu.TPUCompilerParams


def kernel(ids, E):
    my_x = lax.axis_index("x")
    off = my_x * V_LOCAL
    owned = (ids >= off) & (ids < off + V_LOCAL)
    ow = owned.astype(jnp.int32)
    loc = jnp.clip(ids - off, 0, V_LOCAL - 1).astype(jnp.int32)
    cnt = jnp.sum(ow.reshape(CH, R), axis=1).astype(jnp.int32)
    mask = owned.astype(jnp.float32)[:, None]

    def body(loc_ref, ow_ref, cnt_ref, mask_ref, e_ref, out_ref,
             send_buf, recv_buf, gsem, send_sems, recv_sems):
        mx = lax.axis_index("x")
        my = lax.axis_index("y")
        mz = lax.axis_index("z")
        partner = (1 - mx, my, mz)

        barrier_sem = pltpu.get_barrier_semaphore()
        _sem_signal(barrier_sem, inc=1, device_id=partner,
                    device_id_type=pl.DeviceIdType.MESH,)
        _sem_wait(barrier_sem, 1)

        def issue_chunk(c):
            def issue(i, carry):
                @pl.when(ow_ref[i] == 1)
                def _():
                    pltpu.make_async_copy(
                        e_ref.at[pl.ds(loc_ref[i], 1)],
                        out_ref.at[pl.ds(i, 1)],
                        gsem.at[c],
                    ).start()
                return carry

            lax.fori_loop(c * R, (c + 1) * R, issue, 0, unroll=8)

        issue_chunk(0)
        issue_chunk(1)

        rdmas = []
        for c in range(CH):
            cnt_c = cnt_ref[c]
            for k in range(8):
                @pl.when(lax.bitwise_and(lax.shift_right_logical(cnt_c, k), 1) == 1)
                def _(c=c, k=k):
                    pltpu.make_async_copy(
                        e_ref.at[pl.ds(0, 1 << k)],
                        out_ref.at[pl.ds(0, 1 << k)],
                        gsem.at[c],
                    ).wait()

            sl = pl.ds(c * R, R)
            m = jnp.where(mask_ref[sl, :] > 0.0, out_ref[sl, :], 0.0)
            out_ref[sl, :] = m
            send_buf[sl, :] = m.astype(jnp.bfloat16)
            rdma = pltpu.make_async_remote_copy(
                src_ref=send_buf.at[sl],
                dst_ref=recv_buf.at[sl],
                send_sem=send_sems.at[c],
                recv_sem=recv_sems.at[c],
                device_id=partner,
                device_id_type=pl.DeviceIdType.MESH,
            )
            rdma.start()
            rdmas.append(rdma)
            if c + 2 < CH:
                issue_chunk(c + 2)

        for c in range(CH):
            rdmas[c].wait_recv()
            sl = pl.ds(c * R, R)
            out_ref[sl, :] = out_ref[sl, :] + recv_buf[sl, :].astype(jnp.float32)

        for c in range(CH):
            rdmas[c].wait_send()

    return pl.pallas_call(
        body,
        out_shape=jax.ShapeDtypeStruct((T, D), jnp.float32),
        in_specs=[
            pl.BlockSpec(memory_space=pltpu.SMEM),
            pl.BlockSpec(memory_space=pltpu.SMEM),
            pl.BlockSpec(memory_space=pltpu.SMEM),
            pl.BlockSpec(memory_space=pltpu.VMEM),
            pl.BlockSpec(memory_space=pl.ANY),
        ],
        out_specs=pl.BlockSpec(memory_space=pltpu.VMEM),
        scratch_shapes=[
            pltpu.VMEM((T, D), jnp.bfloat16),
            pltpu.VMEM((T, D), jnp.bfloat16),
            pltpu.SemaphoreType.DMA((CH,)),
            pltpu.SemaphoreType.DMA((CH,)),
            pltpu.SemaphoreType.DMA((CH,)),
        ],
        compiler_params=_CompilerParams(collective_id=0),
    )(loc, ow, cnt, mask, E)


# device time: 35379 ns/iter; 1.5221x vs baseline; 1.0061x over previous
import jax
import jax.numpy as jnp
from jax import lax
from jax.experimental import pallas as pl
from jax.experimental.pallas import tpu as pltpu

T = 1024
D = 1024
V_LOCAL = 8192
CH = 8
R = T // CH

_sem_signal = getattr(pl, "semaphore_signal", None) or pltpu.semaphore_signal
_sem_wait = getattr(pl, "semaphore_wait", None) or pltpu.semaphore_wait
_CompilerParams = getattr(pltpu, "CompilerParams", None) or pltpu.TPUCompilerParams


def kernel(ids, E):
    my_x = lax.axis_index("x")
    off = my_x * V_LOCAL
    owned = (ids >= off) & (ids < off + V_LOCAL)
    ow = owned.astype(jnp.int32)
    loc = jnp.clip(ids - off, 0, V_LOCAL - 1).astype(jnp.int32)
    cnt = jnp.sum(ow.reshape(CH, R), axis=1).astype(jnp.int32)
    mask = owned.astype(jnp.float32)[:, None]

    def body(loc_ref, ow_ref, cnt_ref, mask_ref, e_ref, out_ref,
             send_buf, recv_buf, gsem, send_sems, recv_sems):
        mx = lax.axis_index("x")
        my = lax.axis_index("y")
        mz = lax.axis_index("z")
        partner = (1 - mx, my, mz)

        barrier_sem = pltpu.get_barrier_semaphore()
        _sem_signal(barrier_sem, inc=1, device_id=partner,
                    device_id_type=pl.DeviceIdType.MESH,)
        _sem_wait(barrier_sem, 1)

        def issue_chunk(c):
            def issue(i, carry):
                @pl.when(ow_ref[i] == 1)
                def _():
                    pltpu.make_async_copy(
                        e_ref.at[pl.ds(loc_ref[i], 1)],
                        out_ref.at[pl.ds(i, 1)],
                        gsem.at[c],
                    ).start()
                return carry

            lax.fori_loop(c * R, (c + 1) * R, issue, 0, unroll=16)

        issue_chunk(0)
        issue_chunk(1)

        rdmas = []
        for c in range(CH):
            cnt_c = cnt_ref[c]
            for k in range(8):
                @pl.when(lax.bitwise_and(lax.shift_right_logical(cnt_c, k), 1) == 1)
                def _(c=c, k=k):
                    pltpu.make_async_copy(
                        e_ref.at[pl.ds(0, 1 << k)],
                        out_ref.at[pl.ds(0, 1 << k)],
                        gsem.at[c],
                    ).wait()

            sl = pl.ds(c * R, R)
            m = jnp.where(mask_ref[sl, :] > 0.0, out_ref[sl, :], 0.0)
            out_ref[sl, :] = m
            send_buf[sl, :] = m.astype(jnp.bfloat16)
            rdma = pltpu.make_async_remote_copy(
                src_ref=send_buf.at[sl],
                dst_ref=recv_buf.at[sl],
                send_sem=send_sems.at[c],
                recv_sem=recv_sems.at[c],
                device_id=partner,
                device_id_type=pl.DeviceIdType.MESH,
            )
            rdma.start()
            rdmas.append(rdma)
            if c + 2 < CH:
                issue_chunk(c + 2)

        for c in range(CH):
            rdmas[c].wait_recv()
            sl = pl.ds(c * R, R)
            out_ref[sl, :] = out_ref[sl, :] + recv_buf[sl, :].astype(jnp.float32)

        for c in range(CH):
            rdmas[c].wait_send()

    return pl.pallas_call(
        body,
        out_shape=jax.ShapeDtypeStruct((T, D), jnp.float32),
        in_specs=[
            pl.BlockSpec(memory_space=pltpu.SMEM),
            pl.BlockSpec(memory_space=pltpu.SMEM),
            pl.BlockSpec(memory_space=pltpu.SMEM),
            pl.BlockSpec(memory_space=pltpu.VMEM),
            pl.BlockSpec(memory_space=pl.ANY),
        ],
        out_specs=pl.BlockSpec(memory_space=pltpu.VMEM),
        scratch_shapes=[
            pltpu.VMEM((T, D), jnp.bfloat16),
            pltpu.VMEM((T, D), jnp.bfloat16),
            pltpu.SemaphoreType.DMA((CH,)),
            pltpu.SemaphoreType.DMA((CH,)),
            pltpu.SemaphoreType.DMA((CH,)),
        ],
        compiler_params=_CompilerParams(collective_id=0),
    )(loc, ow, cnt, mask, E)


# device time: 35370 ns/iter; 1.5225x vs baseline; 1.0003x over previous
import jax
import jax.numpy as jnp
from jax import lax
from jax.experimental import pallas as pl
from jax.experimental.pallas import tpu as pltpu

T = 1024
D = 1024
V_LOCAL = 8192
CH = 8
R = T // CH

_sem_signal = getattr(pl, "semaphore_signal", None) or pltpu.semaphore_signal
_sem_wait = getattr(pl, "semaphore_wait", None) or pltpu.semaphore_wait
_CompilerParams = getattr(pltpu, "CompilerParams", None) or pltpu.TPUCompilerParams


def kernel(ids, E):
    my_x = lax.axis_index("x")
    off = my_x * V_LOCAL
    owned = (ids >= off) & (ids < off + V_LOCAL)
    ow = owned.astype(jnp.int32)
    loc = jnp.clip(ids - off, 0, V_LOCAL - 1).astype(jnp.int32)
    cnt = jnp.sum(ow.reshape(CH, R), axis=1).astype(jnp.int32)
    mask = owned.astype(jnp.float32)[:, None]

    def body(loc_ref, ow_ref, cnt_ref, mask_ref, e_ref, out_ref,
             send_buf, recv_buf, gsem, send_sems, recv_sems):
        mx = lax.axis_index("x")
        my = lax.axis_index("y")
        mz = lax.axis_index("z")
        partner = (1 - mx, my, mz)

        barrier_sem = pltpu.get_barrier_semaphore()
        _sem_signal(barrier_sem, inc=1, device_id=partner,
                    device_id_type=pl.DeviceIdType.MESH,)
        _sem_wait(barrier_sem, 1)

        def issue_chunk(c):
            def issue(i, carry):
                @pl.when(ow_ref[i] == 1)
                def _():
                    pltpu.make_async_copy(
                        e_ref.at[pl.ds(loc_ref[i], 1)],
                        out_ref.at[pl.ds(i, 1)],
                        gsem.at[c],
                    ).start()
                return carry

            lax.fori_loop(c * R, (c + 1) * R, issue, 0, unroll=16)

        issue_chunk(0)
        issue_chunk(1)

        rdmas = []
        for c in range(CH):
            cnt_c = cnt_ref[c]
            for k in range(8):
                @pl.when(lax.bitwise_and(lax.shift_right_logical(cnt_c, k), 1) == 1)
                def _(c=c, k=k):
                    pltpu.make_async_copy(
                        e_ref.at[pl.ds(0, 1 << k)],
                        out_ref.at[pl.ds(0, 1 << k)],
                        gsem.at[c],
                    ).wait()

            sl = pl.ds(c * R, R)
            m = jnp.where(mask_ref[sl, :] > 0.0, out_ref[sl, :], 0.0)
            out_ref[sl, :] = m
            send_buf[sl, :] = m.astype(jnp.bfloat16)
            rdma = pltpu.make_async_remote_copy(
                src_ref=send_buf.at[sl],
                dst_ref=recv_buf.at[sl],
                send_sem=send_sems.at[c],
                recv_sem=recv_sems.at[c],
                device_id=partner,
                device_id_type=pl.DeviceIdType.MESH,
            )
            rdma.start()
            rdmas.append(rdma)
            if c + 2 < CH:
                issue_chunk(c + 2)
            if c >= 4:
                a = c - 4
                rdmas[a].wait_recv()
                asl = pl.ds(a * R, R)
                out_ref[asl, :] = (
                    out_ref[asl, :] + recv_buf[asl, :].astype(jnp.float32)
                )

        for c in range(CH - 4, CH):
            rdmas[c].wait_recv()
            sl = pl.ds(c * R, R)
            out_ref[sl, :] = out_ref[sl, :] + recv_buf[sl, :].astype(jnp.float32)

        for c in range(CH):
            rdmas[c].wait_send()

    return pl.pallas_call(
        body,
        out_shape=jax.ShapeDtypeStruct((T, D), jnp.float32),
        in_specs=[
            pl.BlockSpec(memory_space=pltpu.SMEM),
            pl.BlockSpec(memory_space=pltpu.SMEM),
            pl.BlockSpec(memory_space=pltpu.SMEM),
            pl.BlockSpec(memory_space=pltpu.VMEM),
            pl.BlockSpec(memory_space=pl.ANY),
        ],
        out_specs=pl.BlockSpec(memory_space=pltpu.VMEM),
        scratch_shapes=[
            pltpu.VMEM((T, D), jnp.bfloat16),
            pltpu.VMEM((T, D), jnp.bfloat16),
            pltpu.SemaphoreType.DMA((CH,)),
            pltpu.SemaphoreType.DMA((CH,)),
            pltpu.SemaphoreType.DMA((CH,)),
        ],
        compiler_params=_CompilerParams(collective_id=0),
    )(loc, ow, cnt, mask, E)
